# baseline (speedup 1.0000x reference)
import functools

import jax
import jax.numpy as jnp
import numpy as np
from jax import lax
from jax.experimental import pallas as pl
from jax.experimental.pallas import tpu as pltpu
from jax.experimental.pallas import tpu_sc as plsc

N_NODES = 10000
N_EDGES = 320000
D_IN = 128
D_OUT = 128

NUM_CORES = 2
NUM_SUBCORES = 16
EDGES_PER_CORE = N_EDGES // NUM_CORES
CHUNK = 128
CHUNKS_PER_TILE = 78
TILE_EDGES = CHUNK * CHUNKS_PER_TILE
LEFTOVER0 = NUM_SUBCORES * TILE_EDGES

ROWS_PER_TILE = 624
TAIL_ROW0 = NUM_SUBCORES * ROWS_PER_TILE
TAIL_ROWS = N_NODES - TAIL_ROW0
STAGE_ROWS = 104
N_STAGE = ROWS_PER_TILE // STAGE_ROWS


_QPERM = np.concatenate([
    32 * g + np.stack([np.arange(16), np.arange(16, 32)], axis=1).reshape(32)
    for g in range(D_OUT // 32)
])


def _matmul_body(x_ref, w_ref, o_ref):
    o_ref[...] = jax.lax.dot_general(
        x_ref[...], w_ref[...], (((1,), (0,)), ((), ())),
        precision=jax.lax.Precision.HIGHEST,
        preferred_element_type=jnp.float32,
    ).astype(jnp.bfloat16)


def _support_packed(x, W):
    return pl.pallas_call(
        _matmul_body,
        grid=(10,),
        in_specs=[
            pl.BlockSpec((N_NODES // 10, D_IN), lambda i: (i, 0)),
            pl.BlockSpec((D_IN, D_OUT), lambda i: (0, 0)),
        ],
        out_specs=pl.BlockSpec((N_NODES // 10, D_OUT), lambda i: (i, 0)),
        out_shape=jax.ShapeDtypeStruct((N_NODES, D_OUT), jnp.bfloat16),
    )(x, W[:, _QPERM])


def _sc_spmm(sup, src, dst, ew):
    mesh = plsc.VectorSubcoreMesh(core_axis_name="c", subcore_axis_name="s")

    @functools.partial(
        pl.kernel,
        mesh=mesh,
        out_type=jax.ShapeDtypeStruct((NUM_CORES, N_NODES, D_OUT), jnp.float32),
        compiler_params=pltpu.CompilerParams(use_tc_tiling_on_sc=False,
                                             needs_layout_passes=False),
        scratch_types=[
            pltpu.VMEM((CHUNK,), jnp.int32),
            pltpu.VMEM((CHUNK,), jnp.int32),
            pltpu.VMEM((CHUNK,), jnp.float32),
            pltpu.VMEM((CHUNK,), jnp.int32),
            pltpu.VMEM((CHUNK,), jnp.int32),
            pltpu.VMEM((CHUNK,), jnp.float32),
            pltpu.VMEM((CHUNK, D_OUT), jnp.bfloat16),
            pltpu.VMEM((CHUNK, D_OUT), jnp.bfloat16),
            pltpu.VMEM((CHUNK, D_OUT), jnp.float32),
            pltpu.VMEM((CHUNK, D_OUT), jnp.float32),
            pltpu.VMEM((2, CHUNK // 2), jnp.int32),
            pltpu.VMEM((2, CHUNK // 2), jnp.int32),
            pltpu.VMEM_SHARED((N_NODES, D_OUT), jnp.float32),
            pltpu.SemaphoreType.DMA,
            pltpu.SemaphoreType.DMA,
            pltpu.SemaphoreType.DMA,
            pltpu.SemaphoreType.DMA,
            pltpu.SemaphoreType.DMA,
            pltpu.SemaphoreType.DMA,
        ],
    )
    def k(sup_hbm, src_hbm, dst_hbm, ew_hbm, out_hbm,
          src_a, dst_a, ew_a, src_b, dst_b, ew_b, grows_a, grows_b,
          frows_a, frows_b, dsc_a, dsc_b, acc, ga, gb, ia, ib, sa, sb):
        cid = lax.axis_index("c")
        sid = lax.axis_index("s")

        corebase = cid * EDGES_PER_CORE
        tilebase = corebase + sid * TILE_EDGES
        npairs = jnp.where(sid == 0, CHUNKS_PER_TILE // 2 + 1,
                           CHUNKS_PER_TILE // 2)
        cmax = 2 * npairs - 1

        def chunk_base(c):
            return jnp.where(
                c < CHUNKS_PER_TILE,
                tilebase + c * CHUNK,
                corebase + LEFTOVER0 + (c - CHUNKS_PER_TILE) * CHUNK)

        def load_idx(c, s_v, d_v, w_v, sem):
            base = chunk_base(c)
            cps = [
                pltpu.async_copy(src_hbm.at[pl.ds(base, CHUNK)], s_v, sem),
                pltpu.async_copy(dst_hbm.at[pl.ds(base, CHUNK)], d_v, sem),
                pltpu.async_copy(ew_hbm.at[pl.ds(base, CHUNK)], w_v, sem),
            ]
            return cps

        def wait_idx(cps):
            for cp in cps:
                cp.wait()

        HALF = CHUNK // 2

        def mul_half(grows_v, frows_v, ew_v, h):
            def mul_group(g16, c2):
                gbase = h * HALF + g16 * 16
                w16 = ew_v[pl.ds(gbase, 16)]
                for r in range(16):
                    e = gbase + r
                    wv = w16[r]
                    for g in range(D_OUT // 32):
                        v32 = grows_v[e, pl.ds(g * 32, 32)]
                        lo, hi = plsc.unpack(
                            v32, format=plsc.PackFormat.INTERLEAVED)
                        frows_v[e, pl.ds(g * 32, 16)] = lo * wv
                        frows_v[e, pl.ds(g * 32 + 16, 16)] = hi * wv
                return c2

            lax.fori_loop(0, HALF // 16, mul_group, 0)

        def scatter_chunk(grows_v, frows_v, ew_v, dst_v, dsc_v, sem):
            for h in range(2):
                mul_half(grows_v, frows_v, ew_v, h)
                for j in range(HALF // 16):
                    dsc_v[h, pl.ds(j * 16, 16)] = (
                        dst_v[pl.ds(h * HALF + j * 16, 16)])
                pltpu.async_copy(frows_v.at[pl.ds(h * HALF, HALF)],
                                 acc.at[dsc_v.at[h]], sem, add=True)

        def wait_scatter(frows_v, dsc_v, sem):
            for h in range(2):
                pltpu.make_async_copy(frows_v.at[pl.ds(h * HALF, HALF)],
                                      acc.at[dsc_v.at[h]], sem).wait()

        wait_idx(load_idx(0, src_a, dst_a, ew_a, ia))
        cp_ga = [pltpu.async_copy(sup_hbm.at[src_a], grows_a, ga)]
        cps_ib = load_idx(1, src_b, dst_b, ew_b, ib)

        zeros16 = jnp.zeros((16,), jnp.float32)

        def zero_body(r, carry):
            for j in range(D_OUT // 16):
                frows_b[r, pl.ds(j * 16, 16)] = zeros16
            return carry

        lax.fori_loop(0, CHUNK, zero_body, 0)
        row0 = sid * ROWS_PER_TILE
        for kk in range(ROWS_PER_TILE // CHUNK + 1):
            nrows = CHUNK if kk < ROWS_PER_TILE // CHUNK else ROWS_PER_TILE % CHUNK
            pltpu.sync_copy(frows_b.at[pl.ds(0, nrows)],
                            acc.at[pl.ds(row0 + kk * CHUNK, nrows)])

        @pl.when(sid == NUM_SUBCORES - 1)
        def _zero_tail():
            pltpu.sync_copy(frows_b.at[pl.ds(0, TAIL_ROWS)],
                            acc.at[pl.ds(TAIL_ROW0, TAIL_ROWS)])

        plsc.subcore_barrier()

        izeros16 = jnp.zeros((16,), jnp.int32)
        for h in range(2):
            for j in range(HALF // 16):
                dsc_b[h, pl.ds(j * 16, 16)] = izeros16
            pltpu.async_copy(frows_b.at[pl.ds(h * HALF, HALF)],
                             acc.at[dsc_b.at[h]], sb, add=True)

        def pair_body(j, carry):
            c_a = 2 * j
            c_b = c_a + 1
            wait_idx(cps_ib)
            wait_scatter(frows_b, dsc_b, sb)
            cp_gb = pltpu.async_copy(sup_hbm.at[src_b], grows_b, gb)
            wait_idx(cp_ga)
            scatter_chunk(grows_a, frows_a, ew_a, dst_a, dsc_a, sa)
            cps_ia = load_idx(jnp.minimum(c_a + 2, cmax), src_a, dst_a,
                              ew_a, ia)
            cp_gb.wait()
            scatter_chunk(grows_b, frows_b, ew_b, dst_b, dsc_b, sb)
            cps_ib2 = load_idx(jnp.minimum(c_b + 2, cmax), src_b, dst_b,
                               ew_b, ib)
            wait_scatter(frows_a, dsc_a, sa)
            wait_idx(cps_ia)
            cp_ga2 = pltpu.async_copy(sup_hbm.at[src_a], grows_a, ga)
            return carry

        lax.fori_loop(0, npairs, pair_body, 0)
        pltpu.make_async_copy(sup_hbm.at[src_a], grows_a, ga).wait()
        wait_scatter(frows_b, dsc_b, sb)
        pltpu.make_async_copy(src_hbm.at[pl.ds(0, CHUNK)], src_b, ib).wait()
        pltpu.make_async_copy(dst_hbm.at[pl.ds(0, CHUNK)], dst_b, ib).wait()
        pltpu.make_async_copy(ew_hbm.at[pl.ds(0, CHUNK)], ew_b, ib).wait()
        plsc.subcore_barrier()

        for kk in range(ROWS_PER_TILE // CHUNK + 1):
            nrows = CHUNK if kk < ROWS_PER_TILE // CHUNK else ROWS_PER_TILE % CHUNK
            r0 = row0 + kk * CHUNK
            pltpu.sync_copy(acc.at[pl.ds(r0, nrows)],
                            frows_a.at[pl.ds(0, nrows)])
            pltpu.sync_copy(frows_a.at[pl.ds(0, nrows)],
                            out_hbm.at[cid].at[pl.ds(r0, nrows)])

        @pl.when(sid == NUM_SUBCORES - 1)
        def _write_tail():
            pltpu.sync_copy(acc.at[pl.ds(TAIL_ROW0, TAIL_ROWS)],
                            frows_a.at[pl.ds(0, TAIL_ROWS)])
            pltpu.sync_copy(
                frows_a.at[pl.ds(0, TAIL_ROWS)],
                out_hbm.at[cid].at[pl.ds(TAIL_ROW0, TAIL_ROWS)])

    return k(sup, src, dst, ew)


def _combine_body(p_ref, o_ref):
    o_ref[...] = p_ref[0] + p_ref[1]


def _combine(partials):
    return pl.pallas_call(
        _combine_body,
        grid=(10,),
        in_specs=[pl.BlockSpec((NUM_CORES, N_NODES // 10, D_OUT),
                               lambda i: (0, i, 0))],
        out_specs=pl.BlockSpec((N_NODES // 10, D_OUT), lambda i: (i, 0)),
        out_shape=jax.ShapeDtypeStruct((N_NODES, D_OUT), jnp.float32),
    )(partials)


def kernel(x, edge_index, edge_weight, W):
    src = edge_index[0].astype(jnp.int32)
    dst = edge_index[1].astype(jnp.int32)
    sup = _support_packed(x, W)
    return _combine(_sc_spmm(sup, src, dst, edge_weight))


# speedup vs baseline: 1.9227x; 1.9227x over previous
import functools

import jax
import jax.numpy as jnp
import numpy as np
from jax import lax
from jax.experimental import pallas as pl
from jax.experimental.pallas import tpu as pltpu
from jax.experimental.pallas import tpu_sc as plsc

N_NODES = 10000
N_EDGES = 320000
D_IN = 128
D_OUT = 128

NUM_CORES = 2
NUM_SUBCORES = 16
EDGES_PER_CORE = N_EDGES // NUM_CORES
CHUNK = 128
CHUNKS_PER_TILE = 78
TILE_EDGES = CHUNK * CHUNKS_PER_TILE
LEFTOVER0 = NUM_SUBCORES * TILE_EDGES

ROWS_PER_TILE = 624
TAIL_ROW0 = NUM_SUBCORES * ROWS_PER_TILE
TAIL_ROWS = N_NODES - TAIL_ROW0
STAGE_ROWS = 104
N_STAGE = ROWS_PER_TILE // STAGE_ROWS


def _matmul_body(x_ref, w_ref, o_ref):
    o_ref[...] = jax.lax.dot_general(
        x_ref[...], w_ref[...], (((1,), (0,)), ((), ())),
        precision=jax.lax.Precision.HIGHEST,
        preferred_element_type=jnp.float32,
    )


def _support(x, W):
    return pl.pallas_call(
        _matmul_body,
        grid=(10,),
        in_specs=[
            pl.BlockSpec((N_NODES // 10, D_IN), lambda i: (i, 0)),
            pl.BlockSpec((D_IN, D_OUT), lambda i: (0, 0)),
        ],
        out_specs=pl.BlockSpec((N_NODES // 10, D_OUT), lambda i: (i, 0)),
        out_shape=jax.ShapeDtypeStruct((N_NODES, D_OUT), jnp.float32),
    )(x, W)


def _sc_spmm(sup, src, dst, ew):
    mesh = plsc.VectorSubcoreMesh(core_axis_name="c", subcore_axis_name="s")

    @functools.partial(
        pl.kernel,
        mesh=mesh,
        out_type=jax.ShapeDtypeStruct((NUM_CORES, N_NODES, D_OUT), jnp.float32),
        scratch_types=[
            pltpu.VMEM((CHUNK,), jnp.int32),
            pltpu.VMEM((CHUNK,), jnp.int32),
            pltpu.VMEM((CHUNK,), jnp.float32),
            pltpu.VMEM((CHUNK,), jnp.int32),
            pltpu.VMEM((CHUNK,), jnp.int32),
            pltpu.VMEM((CHUNK,), jnp.float32),
            pltpu.VMEM((CHUNK, D_OUT), jnp.float32),
            pltpu.VMEM((CHUNK, D_OUT), jnp.float32),
            pltpu.VMEM((2, CHUNK // 2), jnp.int32),
            pltpu.VMEM((2, CHUNK // 2), jnp.int32),
            pltpu.VMEM_SHARED((N_NODES, D_OUT), jnp.float32),
            pltpu.SemaphoreType.DMA,
            pltpu.SemaphoreType.DMA,
            pltpu.SemaphoreType.DMA,
            pltpu.SemaphoreType.DMA,
            pltpu.SemaphoreType.DMA,
            pltpu.SemaphoreType.DMA,
        ],
    )
    def k(sup_hbm, src_hbm, dst_hbm, ew_hbm, out_hbm,
          src_a, dst_a, ew_a, src_b, dst_b, ew_b, frows_a, frows_b,
          dsc_a, dsc_b, acc, ga, gb, ia, ib, sa, sb):
        cid = lax.axis_index("c")
        sid = lax.axis_index("s")

        corebase = cid * EDGES_PER_CORE
        tilebase = corebase + sid * TILE_EDGES
        npairs = jnp.where(sid == 0, CHUNKS_PER_TILE // 2 + 1,
                           CHUNKS_PER_TILE // 2)
        cmax = 2 * npairs - 1

        def chunk_base(c):
            return jnp.where(
                c < CHUNKS_PER_TILE,
                tilebase + c * CHUNK,
                corebase + LEFTOVER0 + (c - CHUNKS_PER_TILE) * CHUNK)

        def load_idx(c, s_v, d_v, w_v, sem):
            base = chunk_base(c)
            cps = [
                pltpu.async_copy(src_hbm.at[pl.ds(base, CHUNK)], s_v, sem),
                pltpu.async_copy(dst_hbm.at[pl.ds(base, CHUNK)], d_v, sem),
                pltpu.async_copy(ew_hbm.at[pl.ds(base, CHUNK)], w_v, sem),
            ]
            return cps

        def wait_idx(cps):
            for cp in cps:
                cp.wait()

        HALF = CHUNK // 2

        def mul_half(frows_v, ew_v, h):
            def mul_group(g16, c2):
                gbase = h * HALF + g16 * 16
                w16 = ew_v[pl.ds(gbase, 16)]
                for r in range(16):
                    e = gbase + r
                    wv = w16[r]
                    for j in range(D_OUT // 16):
                        sl = pl.ds(j * 16, 16)
                        frows_v[e, sl] = frows_v[e, sl] * wv
                return c2

            lax.fori_loop(0, HALF // 16, mul_group, 0)

        def scatter_chunk(frows_v, ew_v, dst_v, dsc_v, sem):
            for h in range(2):
                mul_half(frows_v, ew_v, h)
                for j in range(HALF // 16):
                    dsc_v[h, pl.ds(j * 16, 16)] = (
                        dst_v[pl.ds(h * HALF + j * 16, 16)])
                pltpu.async_copy(frows_v.at[pl.ds(h * HALF, HALF)],
                                 acc.at[dsc_v.at[h]], sem, add=True)

        def wait_scatter(frows_v, dsc_v, sem):
            for h in range(2):
                pltpu.make_async_copy(frows_v.at[pl.ds(h * HALF, HALF)],
                                      acc.at[dsc_v.at[h]], sem).wait()

        wait_idx(load_idx(0, src_a, dst_a, ew_a, ia))
        cp_ga = [pltpu.async_copy(sup_hbm.at[src_a], frows_a, ga)]
        cps_ib = load_idx(1, src_b, dst_b, ew_b, ib)

        zeros16 = jnp.zeros((16,), jnp.float32)

        def zero_body(r, carry):
            for j in range(D_OUT // 16):
                frows_b[r, pl.ds(j * 16, 16)] = zeros16
            return carry

        lax.fori_loop(0, CHUNK, zero_body, 0)
        row0 = sid * ROWS_PER_TILE
        for kk in range(ROWS_PER_TILE // CHUNK + 1):
            nrows = CHUNK if kk < ROWS_PER_TILE // CHUNK else ROWS_PER_TILE % CHUNK
            pltpu.sync_copy(frows_b.at[pl.ds(0, nrows)],
                            acc.at[pl.ds(row0 + kk * CHUNK, nrows)])

        @pl.when(sid == NUM_SUBCORES - 1)
        def _zero_tail():
            pltpu.sync_copy(frows_b.at[pl.ds(0, TAIL_ROWS)],
                            acc.at[pl.ds(TAIL_ROW0, TAIL_ROWS)])

        plsc.subcore_barrier()

        izeros16 = jnp.zeros((16,), jnp.int32)
        for h in range(2):
            for j in range(HALF // 16):
                dsc_b[h, pl.ds(j * 16, 16)] = izeros16
            pltpu.async_copy(frows_b.at[pl.ds(h * HALF, HALF)],
                             acc.at[dsc_b.at[h]], sb, add=True)

        def pair_body(j, carry):
            c_a = 2 * j
            c_b = c_a + 1
            wait_idx(cps_ib)
            wait_scatter(frows_b, dsc_b, sb)
            cp_gb = pltpu.async_copy(sup_hbm.at[src_b], frows_b, gb)
            wait_idx(cp_ga)
            scatter_chunk(frows_a, ew_a, dst_a, dsc_a, sa)
            cps_ia = load_idx(jnp.minimum(c_a + 2, cmax), src_a, dst_a,
                              ew_a, ia)
            cp_gb.wait()
            wait_scatter(frows_a, dsc_a, sa)
            wait_idx(cps_ia)
            cp_ga2 = pltpu.async_copy(sup_hbm.at[src_a], frows_a, ga)
            scatter_chunk(frows_b, ew_b, dst_b, dsc_b, sb)
            cps_ib2 = load_idx(jnp.minimum(c_b + 2, cmax), src_b, dst_b,
                               ew_b, ib)
            return carry

        lax.fori_loop(0, npairs, pair_body, 0)
        pltpu.make_async_copy(sup_hbm.at[src_a], frows_a, ga).wait()
        wait_scatter(frows_b, dsc_b, sb)
        pltpu.make_async_copy(src_hbm.at[pl.ds(0, CHUNK)], src_b, ib).wait()
        pltpu.make_async_copy(dst_hbm.at[pl.ds(0, CHUNK)], dst_b, ib).wait()
        pltpu.make_async_copy(ew_hbm.at[pl.ds(0, CHUNK)], ew_b, ib).wait()
        plsc.subcore_barrier()

        for kk in range(ROWS_PER_TILE // CHUNK + 1):
            nrows = CHUNK if kk < ROWS_PER_TILE // CHUNK else ROWS_PER_TILE % CHUNK
            r0 = row0 + kk * CHUNK
            pltpu.sync_copy(acc.at[pl.ds(r0, nrows)],
                            frows_a.at[pl.ds(0, nrows)])
            pltpu.sync_copy(frows_a.at[pl.ds(0, nrows)],
                            out_hbm.at[cid].at[pl.ds(r0, nrows)])

        @pl.when(sid == NUM_SUBCORES - 1)
        def _write_tail():
            pltpu.sync_copy(acc.at[pl.ds(TAIL_ROW0, TAIL_ROWS)],
                            frows_a.at[pl.ds(0, TAIL_ROWS)])
            pltpu.sync_copy(
                frows_a.at[pl.ds(0, TAIL_ROWS)],
                out_hbm.at[cid].at[pl.ds(TAIL_ROW0, TAIL_ROWS)])

    return k(sup, src, dst, ew)


def _combine_body(p_ref, o_ref):
    o_ref[...] = p_ref[0] + p_ref[1]


def _combine(partials):
    return pl.pallas_call(
        _combine_body,
        grid=(10,),
        in_specs=[pl.BlockSpec((NUM_CORES, N_NODES // 10, D_OUT),
                               lambda i: (0, i, 0))],
        out_specs=pl.BlockSpec((N_NODES // 10, D_OUT), lambda i: (i, 0)),
        out_shape=jax.ShapeDtypeStruct((N_NODES, D_OUT), jnp.float32),
    )(partials)


def kernel(x, edge_index, edge_weight, W):
    src = edge_index[0].astype(jnp.int32)
    dst = edge_index[1].astype(jnp.int32)
    sup = _support(x, W)
    return _combine(_sc_spmm(sup, src, dst, edge_weight))


# speedup vs baseline: 1.9375x; 1.0077x over previous
import functools

import jax
import jax.numpy as jnp
from jax import lax
from jax.experimental import pallas as pl
from jax.experimental.pallas import tpu as pltpu
from jax.experimental.pallas import tpu_sc as plsc

N_NODES = 10000
N_EDGES = 320000
D_IN = 128
D_OUT = 128

NUM_CORES = 2
NUM_SUBCORES = 16
EDGES_PER_CORE = N_EDGES // NUM_CORES
CHUNK = 64
HALF = CHUNK // 2
CHUNKS_PER_TILE = 156
TILE_EDGES = CHUNK * CHUNKS_PER_TILE
LEFTOVER0 = NUM_SUBCORES * TILE_EDGES
MINI = (EDGES_PER_CORE - LEFTOVER0) // NUM_SUBCORES
QUADS = CHUNKS_PER_TILE // 4
NBUF = 4

ROWS_PER_TILE = 624
TAIL_ROW0 = NUM_SUBCORES * ROWS_PER_TILE
TAIL_ROWS = N_NODES - TAIL_ROW0


def _matmul_body(x_ref, w_ref, o_ref):
    o_ref[...] = jax.lax.dot_general(
        x_ref[...], w_ref[...], (((1,), (0,)), ((), ())),
        precision=jax.lax.Precision.HIGHEST,
        preferred_element_type=jnp.float32,
    )


def _support(x, W):
    return pl.pallas_call(
        _matmul_body,
        grid=(10,),
        in_specs=[
            pl.BlockSpec((N_NODES // 10, D_IN), lambda i: (i, 0)),
            pl.BlockSpec((D_IN, D_OUT), lambda i: (0, 0)),
        ],
        out_specs=pl.BlockSpec((N_NODES // 10, D_OUT), lambda i: (i, 0)),
        out_shape=jax.ShapeDtypeStruct((N_NODES, D_OUT), jnp.float32),
    )(x, W)


def _sc_spmm(sup, src, dst, ew):
    mesh = plsc.VectorSubcoreMesh(core_axis_name="c", subcore_axis_name="s")

    @functools.partial(
        pl.kernel,
        mesh=mesh,
        out_type=jax.ShapeDtypeStruct((NUM_CORES, N_NODES, D_OUT), jnp.float32),
        scratch_types=(
            [pltpu.VMEM((CHUNK,), jnp.int32) for _ in range(NBUF)]
            + [pltpu.VMEM((CHUNK,), jnp.int32) for _ in range(NBUF)]
            + [pltpu.VMEM((CHUNK,), jnp.float32) for _ in range(NBUF)]
            + [pltpu.VMEM((CHUNK, D_OUT), jnp.float32) for _ in range(NBUF)]
            + [pltpu.VMEM((2, HALF), jnp.int32) for _ in range(NBUF)]
            + [pltpu.VMEM((MINI,), jnp.int32),
               pltpu.VMEM((MINI,), jnp.int32),
               pltpu.VMEM((MINI,), jnp.float32),
               pltpu.VMEM_SHARED((N_NODES, D_OUT), jnp.float32)]
            + [pltpu.SemaphoreType.DMA for _ in range(3 * NBUF)]
        ),
    )
    def k(sup_hbm, src_hbm, dst_hbm, ew_hbm, out_hbm, *rest):
        srcs = rest[0:4]
        dsts = rest[4:8]
        ews = rest[8:12]
        rows = rest[12:16]
        dscs = rest[16:20]
        msrc, mdst, mew = rest[20], rest[21], rest[22]
        acc = rest[23]
        gsem = rest[24:28]
        isem = rest[28:32]
        ssem = rest[32:36]

        cid = lax.axis_index("c")
        sid = lax.axis_index("s")
        corebase = cid * EDGES_PER_CORE
        tilebase = corebase + sid * TILE_EDGES
        cmax = CHUNKS_PER_TILE - 1

        def load_idx(c, b):
            base = tilebase + c * CHUNK
            pltpu.async_copy(src_hbm.at[pl.ds(base, CHUNK)], srcs[b], isem[b])
            pltpu.async_copy(dst_hbm.at[pl.ds(base, CHUNK)], dsts[b], isem[b])
            pltpu.async_copy(ew_hbm.at[pl.ds(base, CHUNK)], ews[b], isem[b])

        def wait_idx(b):
            pltpu.make_async_copy(
                src_hbm.at[pl.ds(0, CHUNK)], srcs[b], isem[b]).wait()
            pltpu.make_async_copy(
                dst_hbm.at[pl.ds(0, CHUNK)], dsts[b], isem[b]).wait()
            pltpu.make_async_copy(
                ew_hbm.at[pl.ds(0, CHUNK)], ews[b], isem[b]).wait()

        def issue_gather(b):
            pltpu.async_copy(sup_hbm.at[srcs[b]], rows[b], gsem[b])

        def wait_gather(b):
            pltpu.make_async_copy(sup_hbm.at[srcs[b]], rows[b], gsem[b]).wait()

        def wait_scatter(b):
            for h in range(2):
                pltpu.make_async_copy(rows[b].at[pl.ds(h * HALF, HALF)],
                                      acc.at[dscs[b].at[h]], ssem[b]).wait()

        def mul_half(b, h):
            def mul_group(g16, c2):
                gbase = h * HALF + g16 * 16
                w16 = ews[b][pl.ds(gbase, 16)]
                for r in range(16):
                    e = gbase + r
                    wv = w16[r]
                    for j in range(D_OUT // 16):
                        sl = pl.ds(j * 16, 16)
                        rows[b][e, sl] = rows[b][e, sl] * wv
                return c2

            lax.fori_loop(0, HALF // 16, mul_group, 0)

        def scatter_chunk(b):
            for h in range(2):
                mul_half(b, h)
                for j in range(HALF // 16):
                    dscs[b][h, pl.ds(j * 16, 16)] = (
                        dsts[b][pl.ds(h * HALF + j * 16, 16)])
                pltpu.async_copy(rows[b].at[pl.ds(h * HALF, HALF)],
                                 acc.at[dscs[b].at[h]], ssem[b], add=True)

        load_idx(0, 0)
        wait_idx(0)
        issue_gather(0)
        load_idx(1, 1)
        load_idx(2, 2)
        load_idx(3, 3)

        zeros16 = jnp.zeros((16,), jnp.float32)
        izeros16 = jnp.zeros((16,), jnp.int32)

        def zero_body(r, carry):
            for j in range(D_OUT // 16):
                rows[2][r, pl.ds(j * 16, 16)] = zeros16
                rows[3][r, pl.ds(j * 16, 16)] = zeros16
            return carry

        lax.fori_loop(0, CHUNK, zero_body, 0)
        for h in range(2):
            for j in range(HALF // 16):
                dscs[2][h, pl.ds(j * 16, 16)] = izeros16

        row0 = sid * ROWS_PER_TILE
        for kk in range(ROWS_PER_TILE // CHUNK):
            pltpu.sync_copy(rows[2], acc.at[pl.ds(row0 + kk * CHUNK, CHUNK)])
        pltpu.sync_copy(
            rows[2].at[pl.ds(0, ROWS_PER_TILE % CHUNK)],
            acc.at[pl.ds(row0 + ROWS_PER_TILE - ROWS_PER_TILE % CHUNK,
                         ROWS_PER_TILE % CHUNK)])

        @pl.when(sid == NUM_SUBCORES - 1)
        def _zero_tail():
            pltpu.sync_copy(rows[2].at[pl.ds(0, TAIL_ROWS)],
                            acc.at[pl.ds(TAIL_ROW0, TAIL_ROWS)])

        plsc.subcore_barrier()

        for h in range(2):
            pltpu.async_copy(rows[2].at[pl.ds(h * HALF, HALF)],
                             acc.at[dscs[2].at[h]], ssem[2], add=True)
            pltpu.async_copy(rows[3].at[pl.ds(h * HALF, HALF)],
                             acc.at[dscs[2].at[h]], ssem[3], add=True)
        wait_idx(1)
        issue_gather(1)

        def quad_body(t, carry):
            for b in range(NBUF):
                c = 4 * t + b
                wait_gather(b)
                scatter_chunk(b)
                load_idx(jnp.minimum(c + 4, cmax), b)
                b2 = (b + 2) % NBUF
                wait_scatter(b2)
                wait_idx(b2)
                issue_gather(b2)
            return carry

        lax.fori_loop(0, QUADS, quad_body, 0)

        wait_gather(0)
        wait_gather(1)
        wait_scatter(2)
        wait_scatter(3)
        wait_idx(2)
        wait_idx(3)

        mbase = corebase + LEFTOVER0 + sid * MINI
        pltpu.sync_copy(src_hbm.at[pl.ds(mbase, MINI)], msrc)
        pltpu.sync_copy(dst_hbm.at[pl.ds(mbase, MINI)], mdst)
        pltpu.sync_copy(ew_hbm.at[pl.ds(mbase, MINI)], mew)
        pltpu.async_copy(sup_hbm.at[msrc], rows[0].at[pl.ds(0, MINI)],
                         gsem[0]).wait()
        mw16 = mew[...]
        for r in range(MINI):
            wv = mw16[r]
            for j in range(D_OUT // 16):
                sl = pl.ds(j * 16, 16)
                rows[0][r, sl] = rows[0][r, sl] * wv
        pltpu.sync_copy(rows[0].at[pl.ds(0, MINI)], acc.at[mdst], add=True)

        plsc.subcore_barrier()

        for kk in range(ROWS_PER_TILE // CHUNK):
            r0 = row0 + kk * CHUNK
            pltpu.sync_copy(acc.at[pl.ds(r0, CHUNK)], rows[1])
            pltpu.sync_copy(rows[1], out_hbm.at[cid].at[pl.ds(r0, CHUNK)])
        rem = ROWS_PER_TILE % CHUNK
        r9 = row0 + ROWS_PER_TILE - rem
        pltpu.sync_copy(acc.at[pl.ds(r9, rem)], rows[1].at[pl.ds(0, rem)])
        pltpu.sync_copy(rows[1].at[pl.ds(0, rem)],
                        out_hbm.at[cid].at[pl.ds(r9, rem)])

        @pl.when(sid == NUM_SUBCORES - 1)
        def _write_tail():
            pltpu.sync_copy(acc.at[pl.ds(TAIL_ROW0, TAIL_ROWS)],
                            rows[1].at[pl.ds(0, TAIL_ROWS)])
            pltpu.sync_copy(
                rows[1].at[pl.ds(0, TAIL_ROWS)],
                out_hbm.at[cid].at[pl.ds(TAIL_ROW0, TAIL_ROWS)])

    return k(sup, src, dst, ew)


def _combine_body(p_ref, o_ref):
    o_ref[...] = p_ref[0] + p_ref[1]


def _combine(partials):
    return pl.pallas_call(
        _combine_body,
        grid=(10,),
        in_specs=[pl.BlockSpec((NUM_CORES, N_NODES // 10, D_OUT),
                               lambda i: (0, i, 0))],
        out_specs=pl.BlockSpec((N_NODES // 10, D_OUT), lambda i: (i, 0)),
        out_shape=jax.ShapeDtypeStruct((N_NODES, D_OUT), jnp.float32),
    )(partials)


def kernel(x, edge_index, edge_weight, W):
    src = edge_index[0].astype(jnp.int32)
    dst = edge_index[1].astype(jnp.int32)
    sup = _support(x, W)
    return _combine(_sc_spmm(sup, src, dst, edge_weight))


# speedup vs baseline: 2.1557x; 1.1126x over previous
import functools

import jax
import jax.numpy as jnp
from jax import lax
from jax.experimental import pallas as pl
from jax.experimental.pallas import tpu as pltpu
from jax.experimental.pallas import tpu_sc as plsc

N_NODES = 10000
N_EDGES = 320000
D_IN = 128
D_OUT = 128

NUM_CORES = 2
NUM_SUBCORES = 16
EDGES_PER_CORE = N_EDGES // NUM_CORES
CHUNK = 64
HALF = CHUNK // 2
CHUNKS_PER_TILE = 156
TILE_EDGES = CHUNK * CHUNKS_PER_TILE
LEFTOVER0 = NUM_SUBCORES * TILE_EDGES
MINI = (EDGES_PER_CORE - LEFTOVER0) // NUM_SUBCORES
QUADS = CHUNKS_PER_TILE // 4
NBUF = 4

ROWS_PER_TILE = 624
TAIL_ROW0 = NUM_SUBCORES * ROWS_PER_TILE
TAIL_ROWS = N_NODES - TAIL_ROW0


def _matmul_body(x_ref, w_ref, o_ref):
    o_ref[...] = jax.lax.dot_general(
        x_ref[...], w_ref[...], (((1,), (0,)), ((), ())),
        preferred_element_type=jnp.float32,
    )


def _support(x, W):
    return pl.pallas_call(
        _matmul_body,
        grid=(10,),
        in_specs=[
            pl.BlockSpec((N_NODES // 10, D_IN), lambda i: (i, 0)),
            pl.BlockSpec((D_IN, D_OUT), lambda i: (0, 0)),
        ],
        out_specs=pl.BlockSpec((N_NODES // 10, D_OUT), lambda i: (i, 0)),
        out_shape=jax.ShapeDtypeStruct((N_NODES, D_OUT), jnp.float32),
    )(x, W)


def _sc_spmm(sup, ei, ew):
    mesh = plsc.VectorSubcoreMesh(core_axis_name="c", subcore_axis_name="s")

    @functools.partial(
        pl.kernel,
        mesh=mesh,
        out_type=jax.ShapeDtypeStruct((NUM_CORES, N_NODES, D_OUT), jnp.float32),
        scratch_types=(
            [pltpu.VMEM((CHUNK,), jnp.int32) for _ in range(NBUF)]
            + [pltpu.VMEM((CHUNK,), jnp.int32) for _ in range(NBUF)]
            + [pltpu.VMEM((CHUNK,), jnp.float32) for _ in range(NBUF)]
            + [pltpu.VMEM((CHUNK, D_OUT), jnp.float32) for _ in range(NBUF)]
            + [pltpu.VMEM((2, HALF), jnp.int32) for _ in range(NBUF)]
            + [pltpu.VMEM((MINI,), jnp.int32),
               pltpu.VMEM((MINI,), jnp.int32),
               pltpu.VMEM((MINI,), jnp.float32),
               pltpu.VMEM_SHARED((N_NODES, D_OUT), jnp.float32)]
            + [pltpu.SemaphoreType.DMA for _ in range(3 * NBUF)]
        ),
    )
    def k(sup_hbm, ei_hbm, ew_hbm, out_hbm, *rest):
        src_hbm = ei_hbm.at[0]
        dst_hbm = ei_hbm.at[1]
        srcs = rest[0:4]
        dsts = rest[4:8]
        ews = rest[8:12]
        rows = rest[12:16]
        dscs = rest[16:20]
        msrc, mdst, mew = rest[20], rest[21], rest[22]
        acc = rest[23]
        gsem = rest[24:28]
        isem = rest[28:32]
        ssem = rest[32:36]

        cid = lax.axis_index("c")
        sid = lax.axis_index("s")
        corebase = cid * EDGES_PER_CORE
        tilebase = corebase + sid * TILE_EDGES
        cmax = CHUNKS_PER_TILE - 1

        def load_idx(c, b):
            base = tilebase + c * CHUNK
            pltpu.async_copy(src_hbm.at[pl.ds(base, CHUNK)], srcs[b], isem[b])
            pltpu.async_copy(dst_hbm.at[pl.ds(base, CHUNK)], dsts[b], isem[b])
            pltpu.async_copy(ew_hbm.at[pl.ds(base, CHUNK)], ews[b], isem[b])

        def wait_idx(b):
            pltpu.make_async_copy(
                src_hbm.at[pl.ds(0, CHUNK)], srcs[b], isem[b]).wait()
            pltpu.make_async_copy(
                dst_hbm.at[pl.ds(0, CHUNK)], dsts[b], isem[b]).wait()
            pltpu.make_async_copy(
                ew_hbm.at[pl.ds(0, CHUNK)], ews[b], isem[b]).wait()

        def issue_gather(b):
            pltpu.async_copy(sup_hbm.at[srcs[b]], rows[b], gsem[b])

        def wait_gather(b):
            pltpu.make_async_copy(sup_hbm.at[srcs[b]], rows[b], gsem[b]).wait()

        def wait_scatter(b):
            for h in range(2):
                pltpu.make_async_copy(rows[b].at[pl.ds(h * HALF, HALF)],
                                      acc.at[dscs[b].at[h]], ssem[b]).wait()

        def mul_half(b, h):
            def mul_group(g16, c2):
                gbase = h * HALF + g16 * 16
                w16 = ews[b][pl.ds(gbase, 16)]
                for r in range(16):
                    e = gbase + r
                    wv = w16[r]
                    for j in range(D_OUT // 16):
                        sl = pl.ds(j * 16, 16)
                        rows[b][e, sl] = rows[b][e, sl] * wv
                return c2

            lax.fori_loop(0, HALF // 16, mul_group, 0)

        def scatter_chunk(b):
            for h in range(2):
                mul_half(b, h)
                for j in range(HALF // 16):
                    dscs[b][h, pl.ds(j * 16, 16)] = (
                        dsts[b][pl.ds(h * HALF + j * 16, 16)])
                pltpu.async_copy(rows[b].at[pl.ds(h * HALF, HALF)],
                                 acc.at[dscs[b].at[h]], ssem[b], add=True)

        load_idx(0, 0)
        wait_idx(0)
        issue_gather(0)
        load_idx(1, 1)
        load_idx(2, 2)
        load_idx(3, 3)

        zeros16 = jnp.zeros((16,), jnp.float32)
        izeros16 = jnp.zeros((16,), jnp.int32)

        def zero_body(r, carry):
            for j in range(D_OUT // 16):
                rows[2][r, pl.ds(j * 16, 16)] = zeros16
                rows[3][r, pl.ds(j * 16, 16)] = zeros16
            return carry

        lax.fori_loop(0, CHUNK, zero_body, 0)
        for h in range(2):
            for j in range(HALF // 16):
                dscs[2][h, pl.ds(j * 16, 16)] = izeros16

        row0 = sid * ROWS_PER_TILE
        for kk in range(ROWS_PER_TILE // CHUNK):
            pltpu.sync_copy(rows[2], acc.at[pl.ds(row0 + kk * CHUNK, CHUNK)])
        pltpu.sync_copy(
            rows[2].at[pl.ds(0, ROWS_PER_TILE % CHUNK)],
            acc.at[pl.ds(row0 + ROWS_PER_TILE - ROWS_PER_TILE % CHUNK,
                         ROWS_PER_TILE % CHUNK)])

        @pl.when(sid == NUM_SUBCORES - 1)
        def _zero_tail():
            pltpu.sync_copy(rows[2].at[pl.ds(0, TAIL_ROWS)],
                            acc.at[pl.ds(TAIL_ROW0, TAIL_ROWS)])

        plsc.subcore_barrier()

        for h in range(2):
            pltpu.async_copy(rows[2].at[pl.ds(h * HALF, HALF)],
                             acc.at[dscs[2].at[h]], ssem[2], add=True)
            pltpu.async_copy(rows[3].at[pl.ds(h * HALF, HALF)],
                             acc.at[dscs[2].at[h]], ssem[3], add=True)
        wait_idx(1)
        issue_gather(1)

        def quad_body(t, carry):
            for b in range(NBUF):
                c = 4 * t + b
                wait_gather(b)
                scatter_chunk(b)
                load_idx(jnp.minimum(c + 4, cmax), b)
                b2 = (b + 2) % NBUF
                wait_scatter(b2)
                wait_idx(b2)
                issue_gather(b2)
            return carry

        lax.fori_loop(0, QUADS, quad_body, 0)

        wait_gather(0)
        wait_gather(1)
        wait_scatter(2)
        wait_scatter(3)
        wait_idx(2)
        wait_idx(3)

        mbase = corebase + LEFTOVER0 + sid * MINI
        pltpu.sync_copy(src_hbm.at[pl.ds(mbase, MINI)], msrc)
        pltpu.sync_copy(dst_hbm.at[pl.ds(mbase, MINI)], mdst)
        pltpu.sync_copy(ew_hbm.at[pl.ds(mbase, MINI)], mew)
        pltpu.async_copy(sup_hbm.at[msrc], rows[0].at[pl.ds(0, MINI)],
                         gsem[0]).wait()
        mw16 = mew[...]
        for r in range(MINI):
            wv = mw16[r]
            for j in range(D_OUT // 16):
                sl = pl.ds(j * 16, 16)
                rows[0][r, sl] = rows[0][r, sl] * wv
        pltpu.sync_copy(rows[0].at[pl.ds(0, MINI)], acc.at[mdst], add=True)

        plsc.subcore_barrier()

        for kk in range(ROWS_PER_TILE // CHUNK):
            r0 = row0 + kk * CHUNK
            pltpu.sync_copy(acc.at[pl.ds(r0, CHUNK)], rows[1])
            pltpu.sync_copy(rows[1], out_hbm.at[cid].at[pl.ds(r0, CHUNK)])
        rem = ROWS_PER_TILE % CHUNK
        r9 = row0 + ROWS_PER_TILE - rem
        pltpu.sync_copy(acc.at[pl.ds(r9, rem)], rows[1].at[pl.ds(0, rem)])
        pltpu.sync_copy(rows[1].at[pl.ds(0, rem)],
                        out_hbm.at[cid].at[pl.ds(r9, rem)])

        @pl.when(sid == NUM_SUBCORES - 1)
        def _write_tail():
            pltpu.sync_copy(acc.at[pl.ds(TAIL_ROW0, TAIL_ROWS)],
                            rows[1].at[pl.ds(0, TAIL_ROWS)])
            pltpu.sync_copy(
                rows[1].at[pl.ds(0, TAIL_ROWS)],
                out_hbm.at[cid].at[pl.ds(TAIL_ROW0, TAIL_ROWS)])

    return k(sup, ei, ew)


def _combine_body(p_ref, o_ref):
    o_ref[...] = p_ref[0] + p_ref[1]


def _combine(partials):
    return pl.pallas_call(
        _combine_body,
        grid=(10,),
        in_specs=[pl.BlockSpec((NUM_CORES, N_NODES // 10, D_OUT),
                               lambda i: (0, i, 0))],
        out_specs=pl.BlockSpec((N_NODES // 10, D_OUT), lambda i: (i, 0)),
        out_shape=jax.ShapeDtypeStruct((N_NODES, D_OUT), jnp.float32),
    )(partials)


def kernel(x, edge_index, edge_weight, W):
    ei = edge_index.astype(jnp.int32)
    sup = _support(x, W)
    return _combine(_sc_spmm(sup, ei, edge_weight))


# speedup vs baseline: 2.1632x; 1.0035x over previous
import functools

import jax
import jax.numpy as jnp
from jax import lax
from jax.experimental import pallas as pl
from jax.experimental.pallas import tpu as pltpu
from jax.experimental.pallas import tpu_sc as plsc

N_NODES = 10000
N_EDGES = 320000
D_IN = 128
D_OUT = 128

NUM_CORES = 2
NUM_SUBCORES = 16
EDGES_PER_CORE = N_EDGES // NUM_CORES
CHUNK = 64
HALF = CHUNK // 2
CHUNKS_PER_TILE = 156
TILE_EDGES = CHUNK * CHUNKS_PER_TILE
LEFTOVER0 = NUM_SUBCORES * TILE_EDGES
MINI = (EDGES_PER_CORE - LEFTOVER0) // NUM_SUBCORES
QUADS = CHUNKS_PER_TILE // 4
NBUF = 4

ROWS_PER_TILE = 624
TAIL_ROW0 = NUM_SUBCORES * ROWS_PER_TILE
TAIL_ROWS = N_NODES - TAIL_ROW0


def _matmul_body(x_ref, w_ref, o_ref):
    o_ref[...] = jax.lax.dot_general(
        x_ref[...], w_ref[...], (((1,), (0,)), ((), ())),
        preferred_element_type=jnp.float32,
    )


def _support(x, W):
    return pl.pallas_call(
        _matmul_body,
        grid=(10,),
        in_specs=[
            pl.BlockSpec((N_NODES // 10, D_IN), lambda i: (i, 0)),
            pl.BlockSpec((D_IN, D_OUT), lambda i: (0, 0)),
        ],
        out_specs=pl.BlockSpec((N_NODES // 10, D_OUT), lambda i: (i, 0)),
        out_shape=jax.ShapeDtypeStruct((N_NODES, D_OUT), jnp.float32),
    )(x, W)


def _sc_spmm(sup, ei, ew):
    mesh = plsc.VectorSubcoreMesh(core_axis_name="c", subcore_axis_name="s")

    @functools.partial(
        pl.kernel,
        mesh=mesh,
        out_type=jax.ShapeDtypeStruct((NUM_CORES, N_NODES, D_OUT), jnp.float32),
        scratch_types=(
            [pltpu.VMEM((CHUNK,), jnp.int32) for _ in range(NBUF)]
            + [pltpu.VMEM((CHUNK,), jnp.int32) for _ in range(NBUF)]
            + [pltpu.VMEM((CHUNK,), jnp.float32) for _ in range(NBUF)]
            + [pltpu.VMEM((CHUNK, D_OUT), jnp.float32) for _ in range(NBUF)]
            + [pltpu.VMEM((2, HALF), jnp.int32) for _ in range(NBUF)]
            + [pltpu.VMEM((MINI,), jnp.int32),
               pltpu.VMEM((MINI,), jnp.int32),
               pltpu.VMEM((MINI,), jnp.float32),
               pltpu.VMEM_SHARED((N_NODES, D_OUT), jnp.float32)]
            + [pltpu.SemaphoreType.DMA for _ in range(3 * NBUF)]
        ),
    )
    def k(sup_hbm, ei_hbm, ew_hbm, out_hbm, *rest):
        src_hbm = ei_hbm.at[0]
        dst_hbm = ei_hbm.at[1]
        srcs = rest[0:4]
        dsts = rest[4:8]
        ews = rest[8:12]
        rows = rest[12:16]
        dscs = rest[16:20]
        msrc, mdst, mew = rest[20], rest[21], rest[22]
        acc = rest[23]
        gsem = rest[24:28]
        isem = rest[28:32]
        ssem = rest[32:36]

        cid = lax.axis_index("c")
        sid = lax.axis_index("s")
        corebase = cid * EDGES_PER_CORE
        tilebase = corebase + sid * TILE_EDGES
        cmax = CHUNKS_PER_TILE - 1

        def load_idx(c, b):
            base = tilebase + c * CHUNK
            pltpu.async_copy(src_hbm.at[pl.ds(base, CHUNK)], srcs[b], isem[b])
            pltpu.async_copy(dst_hbm.at[pl.ds(base, CHUNK)], dsts[b], isem[b])
            pltpu.async_copy(ew_hbm.at[pl.ds(base, CHUNK)], ews[b], isem[b])

        def wait_idx(b):
            pltpu.make_async_copy(
                src_hbm.at[pl.ds(0, CHUNK)], srcs[b], isem[b]).wait()
            pltpu.make_async_copy(
                dst_hbm.at[pl.ds(0, CHUNK)], dsts[b], isem[b]).wait()
            pltpu.make_async_copy(
                ew_hbm.at[pl.ds(0, CHUNK)], ews[b], isem[b]).wait()

        def issue_gather(b):
            pltpu.async_copy(sup_hbm.at[srcs[b]], rows[b], gsem[b])

        def wait_gather(b):
            pltpu.make_async_copy(sup_hbm.at[srcs[b]], rows[b], gsem[b]).wait()

        def wait_scatter(b):
            for h in range(2):
                pltpu.make_async_copy(rows[b].at[pl.ds(h * HALF, HALF)],
                                      acc.at[dscs[b].at[h]], ssem[b]).wait()

        def mul_half(b, h):
            def mul_group(g16, c2):
                gbase = h * HALF + g16 * 16
                w16 = ews[b][pl.ds(gbase, 16)]
                for r in range(16):
                    e = gbase + r
                    wv = w16[r]
                    for j in range(D_OUT // 16):
                        sl = pl.ds(j * 16, 16)
                        rows[b][e, sl] = rows[b][e, sl] * wv
                return c2

            lax.fori_loop(0, HALF // 16, mul_group, 0)

        def scatter_chunk(b):
            for h in range(2):
                mul_half(b, h)
                for j in range(HALF // 16):
                    dscs[b][h, pl.ds(j * 16, 16)] = (
                        dsts[b][pl.ds(h * HALF + j * 16, 16)])
                pltpu.async_copy(rows[b].at[pl.ds(h * HALF, HALF)],
                                 acc.at[dscs[b].at[h]], ssem[b], add=True)

        load_idx(0, 0)
        wait_idx(0)
        issue_gather(0)
        load_idx(1, 1)
        load_idx(2, 2)
        load_idx(3, 3)

        zeros16 = jnp.zeros((16,), jnp.float32)
        izeros16 = jnp.zeros((16,), jnp.int32)

        def zero_body(r, carry):
            for j in range(D_OUT // 16):
                rows[2][r, pl.ds(j * 16, 16)] = zeros16
                rows[3][r, pl.ds(j * 16, 16)] = zeros16
            return carry

        lax.fori_loop(0, CHUNK, zero_body, 0)
        for h in range(2):
            for j in range(HALF // 16):
                dscs[2][h, pl.ds(j * 16, 16)] = izeros16

        row0 = sid * ROWS_PER_TILE
        for kk in range(ROWS_PER_TILE // CHUNK):
            pltpu.sync_copy(rows[2], acc.at[pl.ds(row0 + kk * CHUNK, CHUNK)])
        pltpu.sync_copy(
            rows[2].at[pl.ds(0, ROWS_PER_TILE % CHUNK)],
            acc.at[pl.ds(row0 + ROWS_PER_TILE - ROWS_PER_TILE % CHUNK,
                         ROWS_PER_TILE % CHUNK)])

        @pl.when(sid == NUM_SUBCORES - 1)
        def _zero_tail():
            pltpu.sync_copy(rows[2].at[pl.ds(0, TAIL_ROWS)],
                            acc.at[pl.ds(TAIL_ROW0, TAIL_ROWS)])

        plsc.subcore_barrier()

        for h in range(2):
            pltpu.async_copy(rows[2].at[pl.ds(h * HALF, HALF)],
                             acc.at[dscs[2].at[h]], ssem[2], add=True)
            pltpu.async_copy(rows[3].at[pl.ds(h * HALF, HALF)],
                             acc.at[dscs[2].at[h]], ssem[3], add=True)
        wait_idx(1)
        issue_gather(1)

        def quad_body(t, carry):
            for b in range(NBUF):
                c = 4 * t + b
                wait_gather(b)
                scatter_chunk(b)
                load_idx(jnp.minimum(c + 4, cmax), b)
                b2 = (b + 2) % NBUF
                wait_scatter(b2)
                wait_idx(b2)
                issue_gather(b2)
            return carry

        lax.fori_loop(0, QUADS, quad_body, 0)

        wait_gather(0)
        wait_gather(1)
        wait_scatter(2)
        wait_scatter(3)
        wait_idx(2)
        wait_idx(3)

        mbase = corebase + LEFTOVER0 + sid * MINI
        pltpu.sync_copy(src_hbm.at[pl.ds(mbase, MINI)], msrc)
        pltpu.sync_copy(dst_hbm.at[pl.ds(mbase, MINI)], mdst)
        pltpu.sync_copy(ew_hbm.at[pl.ds(mbase, MINI)], mew)
        pltpu.async_copy(sup_hbm.at[msrc], rows[0].at[pl.ds(0, MINI)],
                         gsem[0]).wait()
        mw16 = mew[...]
        for r in range(MINI):
            wv = mw16[r]
            for j in range(D_OUT // 16):
                sl = pl.ds(j * 16, 16)
                rows[0][r, sl] = rows[0][r, sl] * wv
        pltpu.sync_copy(rows[0].at[pl.ds(0, MINI)], acc.at[mdst], add=True)

        plsc.subcore_barrier()

        pltpu.sync_copy(acc.at[pl.ds(row0, ROWS_PER_TILE)],
                        out_hbm.at[cid].at[pl.ds(row0, ROWS_PER_TILE)])

        @pl.when(sid == NUM_SUBCORES - 1)
        def _write_tail():
            pltpu.sync_copy(acc.at[pl.ds(TAIL_ROW0, TAIL_ROWS)],
                            out_hbm.at[cid].at[pl.ds(TAIL_ROW0, TAIL_ROWS)])

    return k(sup, ei, ew)


def _combine_body(p_ref, o_ref):
    o_ref[...] = p_ref[0] + p_ref[1]


def _combine(partials):
    return pl.pallas_call(
        _combine_body,
        grid=(10,),
        in_specs=[pl.BlockSpec((NUM_CORES, N_NODES // 10, D_OUT),
                               lambda i: (0, i, 0))],
        out_specs=pl.BlockSpec((N_NODES // 10, D_OUT), lambda i: (i, 0)),
        out_shape=jax.ShapeDtypeStruct((N_NODES, D_OUT), jnp.float32),
    )(partials)


def kernel(x, edge_index, edge_weight, W):
    ei = edge_index.astype(jnp.int32)
    sup = _support(x, W)
    return _combine(_sc_spmm(sup, ei, edge_weight))
